# baseline (speedup 1.0000x reference)
import functools

import jax
import jax.numpy as jnp
from jax import lax
from jax.experimental import pallas as pl
from jax.experimental.pallas import tpu as pltpu
from jax.experimental.pallas import tpu_sc as plsc

_H = 3
_D = 128
_NBIG = 4096
_NCO = 512
_NT = _NBIG + _NCO
_EBIG = 262144 + _NBIG
_ECO = 16384 + _NCO
_ECORE = _EBIG + _ECO
_NSUB = 16
_EPT = _ECORE // _NSUB
_NCHUNK = _EPT // 16
_DEN = _NT * _H
_RSEG = _DEN // _NSUB
_BCH = _NCHUNK // 2


def _prep_body(vd, hd, co, wgv, av_m, wgh, ah_m, wgvh, avh_m,
               wghv, ahv_m, wlv, wlh, sc, sv, sh, vh_out, hht_out, tabt):
    def logits(xv, wg, a_m, col):
        wab = jnp.dot(wg[...], a_m[...], preferred_element_type=jnp.float32)
        t = jnp.dot(xv, wab, preferred_element_type=jnp.float32)
        tabt[:, pl.ds(col, xv.shape[0])] = t.T

    logits(vd[...], wgv, av_m, 0)
    logits(co[...], wgvh, avh_m, _NBIG)
    logits(hd[...], wgh, ah_m, _NT)
    logits(co[...].T, wghv, ahv_m, _NT + _NBIG)

    t = (jnp.dot(vd[...], wlv[...] * sc[...],
                 preferred_element_type=jnp.float32) + sv[...])
    vh_out[...] = jnp.where(t >= 0.0, t, 0.01 * t)
    u = (jnp.dot(hd[...], wlh[...] * sc[...],
                 preferred_element_type=jnp.float32) + sh[...])
    hht_out[...] = jnp.where(u >= 0.0, u, 0.01 * u).T


def _prep(vd, hd, co, wgv, av_m, wgh, ah_m, wgvh, avh_m, wghv, ahv_m,
          wlv, wlh, sc, sv, sh):
    n = vd.shape[0]
    return pl.pallas_call(
        _prep_body,
        out_shape=[
            jax.ShapeDtypeStruct((n, _D), jnp.float32),
            jax.ShapeDtypeStruct((_D, n), jnp.float32),
            jax.ShapeDtypeStruct((8, 2 * _NT), jnp.float32),
        ],
    )(vd, hd, co, wgv, av_m, wgh, ah_m, wgvh, avh_m, wghv, ahv_m,
      wlv, wlh, sc, sv, sh)


def _edges_body(vei, hei, coei, cotei, arb, arc,
                eiv, eih, eivhv, eivhh, src_all, dst_all):
    arb_v = arb[...]
    arc_v = arc[...]
    arc_off = arc_v + _NBIG
    for row in (0, 1):
        vr = vei[row, :]
        hr = hei[row, :]
        cor = coei[row, :]
        cotr = cotei[row, :]
        eiv[row, pl.ds(0, _EBIG - _NBIG)] = vr
        eiv[row, pl.ds(_EBIG - _NBIG, _NBIG)] = arb_v
        eih[row, pl.ds(0, _EBIG - _NBIG)] = hr
        eih[row, pl.ds(_EBIG - _NBIG, _NBIG)] = arb_v
        eivhv[row, pl.ds(0, _ECO - _NCO)] = cotr
        eivhv[row, pl.ds(_ECO - _NCO, _NCO)] = arc_v
        eivhh[row, pl.ds(0, _ECO - _NCO)] = cor
        eivhh[row, pl.ds(_ECO - _NCO, _NCO)] = arc_v
        out = src_all if row == 0 else dst_all
        out[pl.ds(0, _EBIG - _NBIG)] = vr
        out[pl.ds(_EBIG - _NBIG, _NBIG)] = arb_v
        out[pl.ds(_EBIG, _ECO - _NCO)] = cotr + _NBIG
        out[pl.ds(_EBIG + _ECO - _NCO, _NCO)] = arc_off
        out[pl.ds(_ECORE, _EBIG - _NBIG)] = hr
        out[pl.ds(_ECORE + _EBIG - _NBIG, _NBIG)] = arb_v
        out[pl.ds(_ECORE + _EBIG, _ECO - _NCO)] = cor + _NBIG
        out[pl.ds(_ECORE + _EBIG + _ECO - _NCO, _NCO)] = arc_off


def _edges(vei, hei, coei, cotei, arb, arc):
    return pl.pallas_call(
        _edges_body,
        out_shape=[
            jax.ShapeDtypeStruct((2, _EBIG), jnp.int32),
            jax.ShapeDtypeStruct((2, _EBIG), jnp.int32),
            jax.ShapeDtypeStruct((2, _ECO), jnp.int32),
            jax.ShapeDtypeStruct((2, _ECO), jnp.int32),
            jax.ShapeDtypeStruct((2 * _ECORE,), jnp.int32),
            jax.ShapeDtypeStruct((2 * _ECORE,), jnp.int32),
        ],
    )(vei, hei, coei, cotei, arb, arc)


def _bigmm_body(vh_ref, hht_ref, pa_ref, pb_ref, p2_ref):
    t = jnp.dot(vh_ref[...], hht_ref[...], preferred_element_type=jnp.float32)
    pa_ref[...] = t
    pb_ref[...] = t
    p2_ref[...] = t + t


def _bigmm(vh, hht):
    n = vh.shape[0]
    m = hht.shape[1]
    tm = 256
    return pl.pallas_call(
        _bigmm_body,
        grid=(n // tm,),
        in_specs=[
            pl.BlockSpec((tm, _D), lambda i: (i, 0)),
            pl.BlockSpec((_D, m), lambda i: (0, 0)),
        ],
        out_specs=[
            pl.BlockSpec((tm, m), lambda i: (i, 0)),
            pl.BlockSpec((tm, m), lambda i: (i, 0)),
            pl.BlockSpec((tm, m), lambda i: (i, 0)),
        ],
        out_shape=[
            jax.ShapeDtypeStruct((n, m), jnp.float32),
            jax.ShapeDtypeStruct((n, m), jnp.float32),
            jax.ShapeDtypeStruct((n, m), jnp.float32),
        ],
    )(vh, hht)


@functools.cache
def _make_alpha_kernel():
    mesh = plsc.VectorSubcoreMesh(core_axis_name="c", subcore_axis_name="s")
    return functools.partial(
        pl.kernel,
        mesh=mesh,
        compiler_params=pltpu.CompilerParams(needs_layout_passes=False),
        out_type=jax.ShapeDtypeStruct((2 * _ECORE * _H,), jnp.float32),
        scratch_types=[
            pltpu.VMEM((_EPT,), jnp.int32),
            pltpu.VMEM((_EPT,), jnp.int32),
            pltpu.VMEM((_NT * 2 * _H,), jnp.float32),
            pltpu.VMEM((_DEN,), jnp.float32),
            pltpu.VMEM((_BCH * 16 * _H,), jnp.float32),
            pltpu.VMEM((_RSEG,), jnp.float32),
            pltpu.VMEM_SHARED((_NSUB * _DEN,), jnp.float32),
            pltpu.VMEM_SHARED((_DEN,), jnp.float32),
        ],
    )(_alpha_body)


def _alpha_body(src_hbm, dst_hbm, tab_hbm, out_hbm,
                src_v, dst_v, tab_v, den_v, out3_v, tmp_v, stage, final_sh):
    c = lax.axis_index("c")
    s = lax.axis_index("s")
    ebase = c * _ECORE + s * _EPT
    pltpu.sync_copy(src_hbm.at[pl.ds(ebase, _EPT)], src_v)
    pltpu.sync_copy(dst_hbm.at[pl.ds(ebase, _EPT)], dst_v)
    for h in range(2 * _H):
        pltpu.sync_copy(tab_hbm.at[h, pl.ds(c * _NT, _NT)],
                        tab_v.at[pl.ds(h * _NT, _NT)])

    zero16 = jnp.zeros((16,), jnp.float32)

    @plsc.parallel_loop(0, _DEN // 16, unroll=4)
    def _zero(i):
        den_v[pl.ds(i * 16, 16)] = zero16

    @plsc.parallel_loop(0, _NCHUNK, unroll=2)
    def _pass1(i):
        off = pl.multiple_of(i * 16, 16)
        isrc = src_v[pl.ds(off, 16)]
        idst = dst_v[pl.ds(off, 16)]
        i3d = idst * _H
        for h in range(_H):
            a = (plsc.load_gather(tab_v, [isrc + h * _NT])
                 + plsc.load_gather(tab_v, [idst + (_H + h) * _NT]))
            a = jnp.where(a >= 0.0, a, 0.2 * a)
            plsc.addupdate_scatter(den_v, [i3d + h], jnp.exp(a))

    pltpu.sync_copy(den_v, stage.at[pl.ds(s * _DEN, _DEN)])
    plsc.subcore_barrier()

    rbase = s * _RSEG

    def _accrow(r, _):
        @pl.when(r != s)
        def _():
            pltpu.sync_copy(stage.at[pl.ds(r * _DEN + rbase, _RSEG)], tmp_v)

            @plsc.parallel_loop(0, _RSEG // 16, unroll=2)
            def _accvec(j):
                off = pl.multiple_of(j * 16, 16)
                den_v[pl.ds(rbase + off, 16)] = (
                    den_v[pl.ds(rbase + off, 16)] + tmp_v[pl.ds(off, 16)])
        return 0
    lax.fori_loop(0, _NSUB, _accrow, 0)

    @plsc.parallel_loop(0, _RSEG // 16, unroll=2)
    def _recip(j):
        off = pl.multiple_of(j * 16, 16)
        den_v[pl.ds(rbase + off, 16)] = 1.0 / (
            den_v[pl.ds(rbase + off, 16)] + 1e-16)

    pltpu.sync_copy(den_v.at[pl.ds(rbase, _RSEG)],
                    final_sh.at[pl.ds(rbase, _RSEG)])
    plsc.subcore_barrier()
    pltpu.sync_copy(final_sh, den_v)

    lane3 = jnp.arange(16, dtype=jnp.int32) * _H
    for blk in range(2):
        @plsc.parallel_loop(0, _BCH, unroll=2)
        def _pass2(i):
            off = pl.multiple_of((blk * _BCH + i) * 16, 16)
            isrc = src_v[pl.ds(off, 16)]
            idst = dst_v[pl.ds(off, 16)]
            ib = i * (16 * _H) + lane3
            for h in range(_H):
                a = (plsc.load_gather(tab_v, [isrc + h * _NT])
                     + plsc.load_gather(tab_v, [idst + (_H + h) * _NT]))
                a = jnp.where(a >= 0.0, a, 0.2 * a)
                rden = plsc.load_gather(den_v, [idst * _H + h])
                plsc.store_scatter(out3_v, [ib + h], jnp.exp(a) * rden)

        pltpu.sync_copy(
            out3_v,
            out_hbm.at[pl.ds((ebase + blk * _BCH * 16) * _H, _BCH * 16 * _H)])


def _att_mat(att_src, att_dst):
    e8 = jnp.eye(8, dtype=jnp.float32)
    a_s = jnp.einsum("hd,hk->hdk", att_src[0], e8[:_H]).reshape(_H * _D, 8)
    a_d = jnp.einsum("hd,hk->hdk", att_dst[0],
                     e8[_H:2 * _H]).reshape(_H * _D, 8)
    return a_s + a_d


def kernel(virus_data, host_data, coexistence_data, virus_edge_index,
           host_edge_index, coexistence_edge_index, coexistence_edge_index_t,
           virus_edge_weight, host_edge_weight,
           W_gat_v, att_src_v, att_dst_v, b_gat_v,
           W_gat_h, att_src_h, att_dst_h, b_gat_h,
           W_gat_vh, att_src_vh, att_dst_vh, b_gat_vh,
           W_gat_hv, att_src_hv, att_dst_hv, b_gat_hv,
           W_lin_v, b_lin_v, W_lin_h, b_lin_h,
           bn_gamma, bn_beta, bn_mean, bn_var):
    scale = bn_gamma / jnp.sqrt(bn_var + 1e-5)

    am_v = _att_mat(att_src_v, att_dst_v)
    am_h = _att_mat(att_src_h, att_dst_h)
    am_vh = _att_mat(att_src_vh, att_dst_vh)
    am_hv = _att_mat(att_src_hv, att_dst_hv)
    sv = (scale * (b_lin_v - bn_mean) + bn_beta)[None, :]
    sh = (scale * (b_lin_h - bn_mean) + bn_beta)[None, :]

    vh, hht, tabt = _prep(
        virus_data, host_data, coexistence_data,
        W_gat_v, am_v, W_gat_h, am_h, W_gat_vh, am_vh, W_gat_hv, am_hv,
        W_lin_v, W_lin_h, scale[None, :], sv, sh)

    ar_big = jnp.arange(_NBIG, dtype=jnp.int32)
    ar_co = jnp.arange(_NCO, dtype=jnp.int32)
    ei_v, ei_h, ei_vhv, ei_vhh, src_all, dst_all = _edges(
        virus_edge_index, host_edge_index, coexistence_edge_index,
        coexistence_edge_index_t, ar_big, ar_co)

    alpha_flat = _make_alpha_kernel()(src_all, dst_all, tabt)

    alpha_v = alpha_flat[:_EBIG * _H].reshape(_EBIG, _H)
    alpha_vhv = alpha_flat[_EBIG * _H:_ECORE * _H].reshape(_ECO, _H)
    alpha_h = alpha_flat[_ECORE * _H:
                         (_ECORE + _EBIG) * _H].reshape(_EBIG, _H)
    alpha_vhh = alpha_flat[(_ECORE + _EBIG) * _H:].reshape(_ECO, _H)

    P, Pb, P2 = _bigmm(vh, hht)

    return (P, Pb, P2, (ei_v, alpha_v), (ei_h, alpha_h),
            (ei_vhv, alpha_vhv), (ei_vhh, alpha_vhh))


# speedup vs baseline: 5.7911x; 5.7911x over previous
import functools

import jax
import jax.numpy as jnp
from jax import lax
from jax.experimental import pallas as pl
from jax.experimental.pallas import tpu as pltpu
from jax.experimental.pallas import tpu_sc as plsc

_H = 3
_D = 128
_NBIG = 4096
_NCO = 512
_NT = _NBIG + _NCO
_EBIG = 262144 + _NBIG
_ECO = 16384 + _NCO
_ECORE = _EBIG + _ECO
_NSUB = 16
_EPT = _ECORE // _NSUB
_NCHUNK = _EPT // 16
_DEN = _NT * _H
_RSEG = _DEN // _NSUB
_BCH = _NCHUNK // 2


def _prep_body(vd, hd, co, wgv, av_m, wgh, ah_m, wgvh, avh_m,
               wghv, ahv_m, wlv, wlh, sc, sv, sh, vh_out, hht_out, tabt):
    def logits(xv, wg, a_m, col):
        wab = jnp.dot(wg[...], a_m[...], preferred_element_type=jnp.float32)
        t = jnp.dot(xv, wab, preferred_element_type=jnp.float32)
        tabt[:, pl.ds(col, xv.shape[0])] = t.T

    logits(vd[...], wgv, av_m, 0)
    logits(co[...], wgvh, avh_m, _NBIG)
    logits(hd[...], wgh, ah_m, _NT)
    logits(co[...].T, wghv, ahv_m, _NT + _NBIG)

    t = (jnp.dot(vd[...], wlv[...] * sc[...],
                 preferred_element_type=jnp.float32) + sv[...])
    vh_out[...] = jnp.where(t >= 0.0, t, 0.01 * t)
    u = (jnp.dot(hd[...], wlh[...] * sc[...],
                 preferred_element_type=jnp.float32) + sh[...])
    hht_out[...] = jnp.where(u >= 0.0, u, 0.01 * u).T


def _prep(vd, hd, co, wgv, av_m, wgh, ah_m, wgvh, avh_m, wghv, ahv_m,
          wlv, wlh, sc, sv, sh):
    n = vd.shape[0]
    return pl.pallas_call(
        _prep_body,
        out_shape=[
            jax.ShapeDtypeStruct((n, _D), jnp.float32),
            jax.ShapeDtypeStruct((_D, n), jnp.float32),
            jax.ShapeDtypeStruct((8, 2 * _NT), jnp.float32),
        ],
    )(vd, hd, co, wgv, av_m, wgh, ah_m, wgvh, avh_m, wghv, ahv_m,
      wlv, wlh, sc, sv, sh)


def _edges_body(vei, hei, coei, cotei, arb, arc,
                eiv, eih, eivhv, eivhh, src_all, dst_all):
    arb_v = arb[...]
    arc_v = arc[...]
    arc_off = arc_v + _NBIG
    for row in (0, 1):
        vr = vei[row, :]
        hr = hei[row, :]
        cor = coei[row, :]
        cotr = cotei[row, :]
        eiv[row, pl.ds(0, _EBIG - _NBIG)] = vr
        eiv[row, pl.ds(_EBIG - _NBIG, _NBIG)] = arb_v
        eih[row, pl.ds(0, _EBIG - _NBIG)] = hr
        eih[row, pl.ds(_EBIG - _NBIG, _NBIG)] = arb_v
        eivhv[row, pl.ds(0, _ECO - _NCO)] = cotr
        eivhv[row, pl.ds(_ECO - _NCO, _NCO)] = arc_v
        eivhh[row, pl.ds(0, _ECO - _NCO)] = cor
        eivhh[row, pl.ds(_ECO - _NCO, _NCO)] = arc_v
        out = src_all if row == 0 else dst_all
        out[pl.ds(0, _EBIG - _NBIG)] = vr
        out[pl.ds(_EBIG - _NBIG, _NBIG)] = arb_v
        out[pl.ds(_EBIG, _ECO - _NCO)] = cotr + _NBIG
        out[pl.ds(_EBIG + _ECO - _NCO, _NCO)] = arc_off
        out[pl.ds(_ECORE, _EBIG - _NBIG)] = hr
        out[pl.ds(_ECORE + _EBIG - _NBIG, _NBIG)] = arb_v
        out[pl.ds(_ECORE + _EBIG, _ECO - _NCO)] = cor + _NBIG
        out[pl.ds(_ECORE + _EBIG + _ECO - _NCO, _NCO)] = arc_off


def _edges(vei, hei, coei, cotei, arb, arc):
    return pl.pallas_call(
        _edges_body,
        out_shape=[
            jax.ShapeDtypeStruct((2, _EBIG), jnp.int32),
            jax.ShapeDtypeStruct((2, _EBIG), jnp.int32),
            jax.ShapeDtypeStruct((2, _ECO), jnp.int32),
            jax.ShapeDtypeStruct((2, _ECO), jnp.int32),
            jax.ShapeDtypeStruct((2 * _ECORE,), jnp.int32),
            jax.ShapeDtypeStruct((2 * _ECORE,), jnp.int32),
        ],
    )(vei, hei, coei, cotei, arb, arc)


def _bigmm_body(vh_ref, hht_ref, pa_ref, pb_ref, p2_ref):
    t = jnp.dot(vh_ref[...], hht_ref[...], preferred_element_type=jnp.float32)
    pa_ref[...] = t
    pb_ref[...] = t
    p2_ref[...] = t + t


def _bigmm(vh, hht):
    n = vh.shape[0]
    m = hht.shape[1]
    tm = 256
    return pl.pallas_call(
        _bigmm_body,
        grid=(n // tm,),
        in_specs=[
            pl.BlockSpec((tm, _D), lambda i: (i, 0)),
            pl.BlockSpec((_D, m), lambda i: (0, 0)),
        ],
        out_specs=[
            pl.BlockSpec((tm, m), lambda i: (i, 0)),
            pl.BlockSpec((tm, m), lambda i: (i, 0)),
            pl.BlockSpec((tm, m), lambda i: (i, 0)),
        ],
        out_shape=[
            jax.ShapeDtypeStruct((n, m), jnp.float32),
            jax.ShapeDtypeStruct((n, m), jnp.float32),
            jax.ShapeDtypeStruct((n, m), jnp.float32),
        ],
    )(vh, hht)


@functools.cache
def _make_alpha_kernel():
    mesh = plsc.VectorSubcoreMesh(core_axis_name="c", subcore_axis_name="s")
    return functools.partial(
        pl.kernel,
        mesh=mesh,
        compiler_params=pltpu.CompilerParams(needs_layout_passes=False),
        out_type=jax.ShapeDtypeStruct((2 * _ECORE * _H,), jnp.float32),
        scratch_types=[
            pltpu.VMEM((_EPT,), jnp.int32),
            pltpu.VMEM((_EPT,), jnp.int32),
            pltpu.VMEM((_NT * 2 * _H,), jnp.float32),
            pltpu.VMEM((_DEN,), jnp.float32),
            pltpu.VMEM((_EPT,), jnp.float32),
            pltpu.VMEM((_RSEG,), jnp.float32),
            pltpu.VMEM_SHARED((_NSUB * _DEN,), jnp.float32),
            pltpu.VMEM_SHARED((_DEN,), jnp.float32),
        ],
    )(_alpha_body)


def _alpha_body(src_hbm, dst_hbm, tab_hbm, out_hbm,
                src_v, dst_v, tab_v, den_v, out3_v, tmp_v, stage, final_sh):
    c = lax.axis_index("c")
    s = lax.axis_index("s")
    ebase = c * _ECORE + s * _EPT
    pltpu.sync_copy(src_hbm.at[pl.ds(ebase, _EPT)], src_v)
    pltpu.sync_copy(dst_hbm.at[pl.ds(ebase, _EPT)], dst_v)
    for h in range(2 * _H):
        pltpu.sync_copy(tab_hbm.at[h, pl.ds(c * _NT, _NT)],
                        tab_v.at[pl.ds(h * _NT, _NT)])

    zero16 = jnp.zeros((16,), jnp.float32)

    @plsc.parallel_loop(0, _DEN // 16, unroll=4)
    def _zero(i):
        den_v[pl.ds(i * 16, 16)] = zero16

    @plsc.parallel_loop(0, _NCHUNK, unroll=2)
    def _pass1(i):
        off = pl.multiple_of(i * 16, 16)
        isrc = src_v[pl.ds(off, 16)]
        idst = dst_v[pl.ds(off, 16)]
        i3d = idst * _H
        for h in range(_H):
            a = (plsc.load_gather(tab_v, [isrc + h * _NT])
                 + plsc.load_gather(tab_v, [idst + (_H + h) * _NT]))
            a = jnp.where(a >= 0.0, a, 0.2 * a)
            plsc.addupdate_scatter(den_v, [i3d + h], jnp.exp(a))

    pltpu.sync_copy(den_v, stage.at[pl.ds(s * _DEN, _DEN)])
    plsc.subcore_barrier()

    rbase = s * _RSEG

    def _accrow(r, _):
        @pl.when(r != s)
        def _():
            pltpu.sync_copy(stage.at[pl.ds(r * _DEN + rbase, _RSEG)], tmp_v)

            @plsc.parallel_loop(0, _RSEG // 16, unroll=2)
            def _accvec(j):
                off = pl.multiple_of(j * 16, 16)
                den_v[pl.ds(rbase + off, 16)] = (
                    den_v[pl.ds(rbase + off, 16)] + tmp_v[pl.ds(off, 16)])
        return 0
    lax.fori_loop(0, _NSUB, _accrow, 0)

    @plsc.parallel_loop(0, _RSEG // 16, unroll=2)
    def _recip(j):
        off = pl.multiple_of(j * 16, 16)
        den_v[pl.ds(rbase + off, 16)] = 1.0 / (
            den_v[pl.ds(rbase + off, 16)] + 1e-16)

    pltpu.sync_copy(den_v.at[pl.ds(rbase, _RSEG)],
                    final_sh.at[pl.ds(rbase, _RSEG)])
    plsc.subcore_barrier()
    pltpu.sync_copy(final_sh, den_v)

    for h in range(_H):
        @plsc.parallel_loop(0, _NCHUNK, unroll=2)
        def _pass2(i):
            off = pl.multiple_of(i * 16, 16)
            isrc = src_v[pl.ds(off, 16)]
            idst = dst_v[pl.ds(off, 16)]
            a = (plsc.load_gather(tab_v, [isrc + h * _NT])
                 + plsc.load_gather(tab_v, [idst + (_H + h) * _NT]))
            a = jnp.where(a >= 0.0, a, 0.2 * a)
            rden = plsc.load_gather(den_v, [idst * _H + h])
            out3_v[pl.ds(off, 16)] = jnp.exp(a) * rden

        pltpu.sync_copy(
            out3_v,
            out_hbm.at[pl.ds((c * _H + h) * _ECORE + s * _EPT, _EPT)])


def _att_mat(att_src, att_dst):
    e8 = jnp.eye(8, dtype=jnp.float32)
    a_s = jnp.einsum("hd,hk->hdk", att_src[0], e8[:_H]).reshape(_H * _D, 8)
    a_d = jnp.einsum("hd,hk->hdk", att_dst[0],
                     e8[_H:2 * _H]).reshape(_H * _D, 8)
    return a_s + a_d


def kernel(virus_data, host_data, coexistence_data, virus_edge_index,
           host_edge_index, coexistence_edge_index, coexistence_edge_index_t,
           virus_edge_weight, host_edge_weight,
           W_gat_v, att_src_v, att_dst_v, b_gat_v,
           W_gat_h, att_src_h, att_dst_h, b_gat_h,
           W_gat_vh, att_src_vh, att_dst_vh, b_gat_vh,
           W_gat_hv, att_src_hv, att_dst_hv, b_gat_hv,
           W_lin_v, b_lin_v, W_lin_h, b_lin_h,
           bn_gamma, bn_beta, bn_mean, bn_var):
    scale = bn_gamma / jnp.sqrt(bn_var + 1e-5)

    am_v = _att_mat(att_src_v, att_dst_v)
    am_h = _att_mat(att_src_h, att_dst_h)
    am_vh = _att_mat(att_src_vh, att_dst_vh)
    am_hv = _att_mat(att_src_hv, att_dst_hv)
    sv = (scale * (b_lin_v - bn_mean) + bn_beta)[None, :]
    sh = (scale * (b_lin_h - bn_mean) + bn_beta)[None, :]

    vh, hht, tabt = _prep(
        virus_data, host_data, coexistence_data,
        W_gat_v, am_v, W_gat_h, am_h, W_gat_vh, am_vh, W_gat_hv, am_hv,
        W_lin_v, W_lin_h, scale[None, :], sv, sh)

    ar_big = jnp.arange(_NBIG, dtype=jnp.int32)
    ar_co = jnp.arange(_NCO, dtype=jnp.int32)
    ei_v, ei_h, ei_vhv, ei_vhh, src_all, dst_all = _edges(
        virus_edge_index, host_edge_index, coexistence_edge_index,
        coexistence_edge_index_t, ar_big, ar_co)

    alpha_flat = _make_alpha_kernel()(src_all, dst_all, tabt)
    alpha_all = alpha_flat.reshape(2, _H, _ECORE)

    alpha_v = alpha_all[0, :, :_EBIG].T
    alpha_vhv = alpha_all[0, :, _EBIG:].T
    alpha_h = alpha_all[1, :, :_EBIG].T
    alpha_vhh = alpha_all[1, :, _EBIG:].T

    P, Pb, P2 = _bigmm(vh, hht)

    return (P, Pb, P2, (ei_v, alpha_v), (ei_h, alpha_h),
            (ei_vhv, alpha_vhv), (ei_vhh, alpha_vhh))


# speedup vs baseline: 6.0258x; 1.0405x over previous
import functools

import jax
import jax.numpy as jnp
from jax import lax
from jax.experimental import pallas as pl
from jax.experimental.pallas import tpu as pltpu
from jax.experimental.pallas import tpu_sc as plsc

_H = 3
_D = 128
_NBIG = 4096
_NCO = 512
_NT = _NBIG + _NCO
_EBIG = 262144 + _NBIG
_ECO = 16384 + _NCO
_ECORE = _EBIG + _ECO
_NSUB = 16
_EPT = _ECORE // _NSUB
_NCHUNK = _EPT // 16
_DEN = _NT * _H
_RSEG = _DEN // _NSUB
_BCH = _NCHUNK // 2


def _prep_body(vd, hd, co, wgv, av_m, wgh, ah_m, wgvh, avh_m,
               wghv, ahv_m, wlv, wlh, sc, sv, sh, vh_out, hht_out, tabt):
    def logits(xv, wg, a_m, col):
        wab = jnp.dot(wg[...], a_m[...], preferred_element_type=jnp.float32)
        t = jnp.dot(xv, wab, preferred_element_type=jnp.float32)
        tabt[:, pl.ds(col, xv.shape[0])] = t.T

    logits(vd[...], wgv, av_m, 0)
    logits(co[...], wgvh, avh_m, _NBIG)
    logits(hd[...], wgh, ah_m, _NT)
    logits(co[...].T, wghv, ahv_m, _NT + _NBIG)

    t = (jnp.dot(vd[...], wlv[...] * sc[...],
                 preferred_element_type=jnp.float32) + sv[...])
    vh_out[...] = jnp.where(t >= 0.0, t, 0.01 * t)
    u = (jnp.dot(hd[...], wlh[...] * sc[...],
                 preferred_element_type=jnp.float32) + sh[...])
    hht_out[...] = jnp.where(u >= 0.0, u, 0.01 * u).T


def _prep(vd, hd, co, wgv, av_m, wgh, ah_m, wgvh, avh_m, wghv, ahv_m,
          wlv, wlh, sc, sv, sh):
    n = vd.shape[0]
    return pl.pallas_call(
        _prep_body,
        out_shape=[
            jax.ShapeDtypeStruct((n, _D), jnp.float32),
            jax.ShapeDtypeStruct((_D, n), jnp.float32),
            jax.ShapeDtypeStruct((8, 2 * _NT), jnp.float32),
        ],
    )(vd, hd, co, wgv, av_m, wgh, ah_m, wgvh, avh_m, wghv, ahv_m,
      wlv, wlh, sc, sv, sh)


def _edges_body(vei, hei, coei, cotei, arb, arc,
                eiv, eih, eivhv, eivhh, src_all, dst_all):
    arb_v = arb[...]
    arc_v = arc[...]
    arc_off = arc_v + _NBIG
    for row in (0, 1):
        vr = vei[row, :]
        hr = hei[row, :]
        cor = coei[row, :]
        cotr = cotei[row, :]
        eiv[row, pl.ds(0, _EBIG - _NBIG)] = vr
        eiv[row, pl.ds(_EBIG - _NBIG, _NBIG)] = arb_v
        eih[row, pl.ds(0, _EBIG - _NBIG)] = hr
        eih[row, pl.ds(_EBIG - _NBIG, _NBIG)] = arb_v
        eivhv[row, pl.ds(0, _ECO - _NCO)] = cotr
        eivhv[row, pl.ds(_ECO - _NCO, _NCO)] = arc_v
        eivhh[row, pl.ds(0, _ECO - _NCO)] = cor
        eivhh[row, pl.ds(_ECO - _NCO, _NCO)] = arc_v
        out = src_all if row == 0 else dst_all
        out[pl.ds(0, _EBIG - _NBIG)] = vr
        out[pl.ds(_EBIG - _NBIG, _NBIG)] = arb_v
        out[pl.ds(_EBIG, _ECO - _NCO)] = cotr + _NBIG
        out[pl.ds(_EBIG + _ECO - _NCO, _NCO)] = arc_off
        out[pl.ds(_ECORE, _EBIG - _NBIG)] = hr
        out[pl.ds(_ECORE + _EBIG - _NBIG, _NBIG)] = arb_v
        out[pl.ds(_ECORE + _EBIG, _ECO - _NCO)] = cor + _NBIG
        out[pl.ds(_ECORE + _EBIG + _ECO - _NCO, _NCO)] = arc_off


def _edges(vei, hei, coei, cotei, arb, arc):
    return pl.pallas_call(
        _edges_body,
        out_shape=[
            jax.ShapeDtypeStruct((2, _EBIG), jnp.int32),
            jax.ShapeDtypeStruct((2, _EBIG), jnp.int32),
            jax.ShapeDtypeStruct((2, _ECO), jnp.int32),
            jax.ShapeDtypeStruct((2, _ECO), jnp.int32),
            jax.ShapeDtypeStruct((2 * _ECORE,), jnp.int32),
            jax.ShapeDtypeStruct((2 * _ECORE,), jnp.int32),
        ],
    )(vei, hei, coei, cotei, arb, arc)


def _bigmm_body(vh_ref, hht_ref, pa_ref, pb_ref, p2_ref):
    t = jnp.dot(vh_ref[...], hht_ref[...], preferred_element_type=jnp.float32)
    pa_ref[...] = t
    pb_ref[...] = t
    p2_ref[...] = t + t


def _bigmm(vh, hht):
    n = vh.shape[0]
    m = hht.shape[1]
    tm = 256
    return pl.pallas_call(
        _bigmm_body,
        grid=(n // tm,),
        in_specs=[
            pl.BlockSpec((tm, _D), lambda i: (i, 0)),
            pl.BlockSpec((_D, m), lambda i: (0, 0)),
        ],
        out_specs=[
            pl.BlockSpec((tm, m), lambda i: (i, 0)),
            pl.BlockSpec((tm, m), lambda i: (i, 0)),
            pl.BlockSpec((tm, m), lambda i: (i, 0)),
        ],
        out_shape=[
            jax.ShapeDtypeStruct((n, m), jnp.float32),
            jax.ShapeDtypeStruct((n, m), jnp.float32),
            jax.ShapeDtypeStruct((n, m), jnp.float32),
        ],
    )(vh, hht)


@functools.cache
def _make_alpha_kernel():
    mesh = plsc.VectorSubcoreMesh(core_axis_name="c", subcore_axis_name="s")
    return functools.partial(
        pl.kernel,
        mesh=mesh,
        compiler_params=pltpu.CompilerParams(needs_layout_passes=False),
        out_type=jax.ShapeDtypeStruct((2 * _ECORE * _H,), jnp.float32),
        scratch_types=[
            pltpu.VMEM((_EPT,), jnp.int32),
            pltpu.VMEM((_EPT,), jnp.int32),
            pltpu.VMEM((_NT * 2 * _H,), jnp.float32),
            pltpu.VMEM((_DEN,), jnp.float32),
            pltpu.VMEM((_EPT,), jnp.float32),
            pltpu.VMEM((_RSEG,), jnp.float32),
            pltpu.VMEM_SHARED((_NSUB * _DEN,), jnp.float32),
            pltpu.VMEM_SHARED((_DEN,), jnp.float32),
        ],
    )(_alpha_body)


def _alpha_body(src_hbm, dst_hbm, tab_hbm, out_hbm,
                src_v, dst_v, tab_v, den_v, out3_v, tmp_v, stage, final_sh):
    c = lax.axis_index("c")
    s = lax.axis_index("s")
    ebase = c * _ECORE + s * _EPT
    pltpu.sync_copy(src_hbm.at[pl.ds(ebase, _EPT)], src_v)
    pltpu.sync_copy(dst_hbm.at[pl.ds(ebase, _EPT)], dst_v)
    for h in range(2 * _H):
        pltpu.sync_copy(tab_hbm.at[h, pl.ds(c * _NT, _NT)],
                        tab_v.at[pl.ds(h * _NT, _NT)])

    zero16 = jnp.zeros((16,), jnp.float32)

    @plsc.parallel_loop(0, _DEN // 16, unroll=4)
    def _zero(i):
        den_v[pl.ds(i * 16, 16)] = zero16

    @plsc.parallel_loop(0, _NCHUNK, unroll=2)
    def _pass1(i):
        off = pl.multiple_of(i * 16, 16)
        isrc = src_v[pl.ds(off, 16)]
        idst = dst_v[pl.ds(off, 16)]
        i3d = idst * _H
        for h in range(_H):
            a = (plsc.load_gather(tab_v, [isrc + h * _NT])
                 + plsc.load_gather(tab_v, [idst + (_H + h) * _NT]))
            a = jnp.where(a >= 0.0, a, 0.2 * a)
            plsc.addupdate_scatter(den_v, [i3d + h], jnp.exp(a))

    pltpu.sync_copy(den_v, stage.at[pl.ds(s * _DEN, _DEN)])
    plsc.subcore_barrier()

    rbase = s * _RSEG

    def _accrow(r, _):
        @pl.when(r != s)
        def _():
            pltpu.sync_copy(stage.at[pl.ds(r * _DEN + rbase, _RSEG)], tmp_v)

            @plsc.parallel_loop(0, _RSEG // 16, unroll=2)
            def _accvec(j):
                off = pl.multiple_of(j * 16, 16)
                den_v[pl.ds(rbase + off, 16)] = (
                    den_v[pl.ds(rbase + off, 16)] + tmp_v[pl.ds(off, 16)])
        return 0
    lax.fori_loop(0, _NSUB, _accrow, 0)

    @plsc.parallel_loop(0, _RSEG // 16, unroll=2)
    def _recip(j):
        off = pl.multiple_of(j * 16, 16)
        den_v[pl.ds(rbase + off, 16)] = 1.0 / (
            den_v[pl.ds(rbase + off, 16)] + 1e-16)

    pltpu.sync_copy(den_v.at[pl.ds(rbase, _RSEG)],
                    final_sh.at[pl.ds(rbase, _RSEG)])
    plsc.subcore_barrier()
    pltpu.sync_copy(final_sh, den_v)

    for h in range(_H):
        @plsc.parallel_loop(0, _NCHUNK, unroll=2)
        def _pass2(i):
            off = pl.multiple_of(i * 16, 16)
            isrc = src_v[pl.ds(off, 16)]
            idst = dst_v[pl.ds(off, 16)]
            a = (plsc.load_gather(tab_v, [isrc + h * _NT])
                 + plsc.load_gather(tab_v, [idst + (_H + h) * _NT]))
            a = jnp.where(a >= 0.0, a, 0.2 * a)
            rden = plsc.load_gather(den_v, [idst * _H + h])
            out3_v[pl.ds(off, 16)] = jnp.exp(a) * rden

        pltpu.sync_copy(
            out3_v,
            out_hbm.at[pl.ds((c * _H + h) * _ECORE + s * _EPT, _EPT)])


def _att_mat(att_src, att_dst):
    e8 = jnp.eye(8, dtype=jnp.float32)
    a_s = jnp.einsum("hd,hk->hdk", att_src[0], e8[:_H]).reshape(_H * _D, 8)
    a_d = jnp.einsum("hd,hk->hdk", att_dst[0],
                     e8[_H:2 * _H]).reshape(_H * _D, 8)
    return a_s + a_d


def kernel(virus_data, host_data, coexistence_data, virus_edge_index,
           host_edge_index, coexistence_edge_index, coexistence_edge_index_t,
           virus_edge_weight, host_edge_weight,
           W_gat_v, att_src_v, att_dst_v, b_gat_v,
           W_gat_h, att_src_h, att_dst_h, b_gat_h,
           W_gat_vh, att_src_vh, att_dst_vh, b_gat_vh,
           W_gat_hv, att_src_hv, att_dst_hv, b_gat_hv,
           W_lin_v, b_lin_v, W_lin_h, b_lin_h,
           bn_gamma, bn_beta, bn_mean, bn_var):
    scale = bn_gamma / jnp.sqrt(bn_var + 1e-5)

    am_v = _att_mat(att_src_v, att_dst_v)
    am_h = _att_mat(att_src_h, att_dst_h)
    am_vh = _att_mat(att_src_vh, att_dst_vh)
    am_hv = _att_mat(att_src_hv, att_dst_hv)
    sv = (scale * (b_lin_v - bn_mean) + bn_beta)[None, :]
    sh = (scale * (b_lin_h - bn_mean) + bn_beta)[None, :]

    vh, hht, tabt = _prep(
        virus_data, host_data, coexistence_data,
        W_gat_v, am_v, W_gat_h, am_h, W_gat_vh, am_vh, W_gat_hv, am_hv,
        W_lin_v, W_lin_h, scale[None, :], sv, sh)

    ar_big = jnp.arange(_NBIG, dtype=jnp.int32)
    ar_co = jnp.arange(_NCO, dtype=jnp.int32)
    ei_v, ei_h, ei_vhv, ei_vhh, src_all, dst_all = _edges(
        virus_edge_index, host_edge_index, coexistence_edge_index,
        coexistence_edge_index_t, ar_big, ar_co)

    alpha_flat = _make_alpha_kernel()(src_all, dst_all, tabt)

    def _planes(core, lo, hi):
        return jnp.stack(
            [alpha_flat[(core * _H + h) * _ECORE + lo:
                        (core * _H + h) * _ECORE + hi] for h in range(_H)],
            axis=1)

    alpha_v = _planes(0, 0, _EBIG)
    alpha_vhv = _planes(0, _EBIG, _ECORE)
    alpha_h = _planes(1, 0, _EBIG)
    alpha_vhh = _planes(1, _EBIG, _ECORE)

    P, Pb, P2 = _bigmm(vh, hht)

    return (P, Pb, P2, (ei_v, alpha_v), (ei_h, alpha_h),
            (ei_vhv, alpha_vhv), (ei_vhh, alpha_vhh))


# speedup vs baseline: 6.0548x; 1.0048x over previous
import functools

import jax
import jax.numpy as jnp
from jax import lax
from jax.experimental import pallas as pl
from jax.experimental.pallas import tpu as pltpu
from jax.experimental.pallas import tpu_sc as plsc

_H = 3
_D = 128
_NBIG = 4096
_NCO = 512
_NT = _NBIG + _NCO
_EBIG = 262144 + _NBIG
_ECO = 16384 + _NCO
_ECORE = _EBIG + _ECO
_NSUB = 16
_EPT = _ECORE // _NSUB
_NCHUNK = _EPT // 16
_DEN = _NT * _H
_RSEG = _DEN // _NSUB
_BCH = _NCHUNK // 2


def _prep_body(vd, hd, co, wgv, av_m, wgh, ah_m, wgvh, avh_m,
               wghv, ahv_m, wlv, wlh, sc, blv, blh, bmean, bbeta,
               vh_out, hht_out, tabt):
    def logits(xv, wg, a_m, col):
        wab = jnp.dot(wg[...], a_m[...], preferred_element_type=jnp.float32)
        t = jnp.dot(xv, wab, preferred_element_type=jnp.float32)
        tabt[:, pl.ds(col, xv.shape[0])] = t.T

    logits(vd[...], wgv, av_m, 0)
    logits(co[...], wgvh, avh_m, _NBIG)
    logits(hd[...], wgh, ah_m, _NT)
    logits(co[...].T, wghv, ahv_m, _NT + _NBIG)

    scv = sc[...]
    sv = scv * (blv[...] - bmean[...]) + bbeta[...]
    sh = scv * (blh[...] - bmean[...]) + bbeta[...]
    t = (jnp.dot(vd[...], wlv[...] * scv,
                 preferred_element_type=jnp.float32) + sv)
    vh_out[...] = jnp.where(t >= 0.0, t, 0.01 * t)
    u = (jnp.dot(hd[...], wlh[...] * scv,
                 preferred_element_type=jnp.float32) + sh)
    hht_out[...] = jnp.where(u >= 0.0, u, 0.01 * u).T


def _prep(vd, hd, co, wgv, av_m, wgh, ah_m, wgvh, avh_m, wghv, ahv_m,
          wlv, wlh, sc, blv, blh, bmean, bbeta):
    n = vd.shape[0]
    return pl.pallas_call(
        _prep_body,
        out_shape=[
            jax.ShapeDtypeStruct((n, _D), jnp.float32),
            jax.ShapeDtypeStruct((_D, n), jnp.float32),
            jax.ShapeDtypeStruct((8, 2 * _NT), jnp.float32),
        ],
    )(vd, hd, co, wgv, av_m, wgh, ah_m, wgvh, avh_m, wghv, ahv_m,
      wlv, wlh, sc, blv, blh, bmean, bbeta)


def _edges_body(vei, hei, coei, cotei,
                eiv, eih, eivhv, eivhh, src_all, dst_all):
    arb_v = lax.broadcasted_iota(jnp.int32, (1, _NBIG), 1)[0]
    arc_v = lax.broadcasted_iota(jnp.int32, (1, _NCO), 1)[0]
    arc_off = arc_v + _NBIG
    for row in (0, 1):
        vr = vei[row, :]
        hr = hei[row, :]
        cor = coei[row, :]
        cotr = cotei[row, :]
        eiv[row, pl.ds(0, _EBIG - _NBIG)] = vr
        eiv[row, pl.ds(_EBIG - _NBIG, _NBIG)] = arb_v
        eih[row, pl.ds(0, _EBIG - _NBIG)] = hr
        eih[row, pl.ds(_EBIG - _NBIG, _NBIG)] = arb_v
        eivhv[row, pl.ds(0, _ECO - _NCO)] = cotr
        eivhv[row, pl.ds(_ECO - _NCO, _NCO)] = arc_v
        eivhh[row, pl.ds(0, _ECO - _NCO)] = cor
        eivhh[row, pl.ds(_ECO - _NCO, _NCO)] = arc_v
        out = src_all if row == 0 else dst_all
        out[pl.ds(0, _EBIG - _NBIG)] = vr
        out[pl.ds(_EBIG - _NBIG, _NBIG)] = arb_v
        out[pl.ds(_EBIG, _ECO - _NCO)] = cotr + _NBIG
        out[pl.ds(_EBIG + _ECO - _NCO, _NCO)] = arc_off
        out[pl.ds(_ECORE, _EBIG - _NBIG)] = hr
        out[pl.ds(_ECORE + _EBIG - _NBIG, _NBIG)] = arb_v
        out[pl.ds(_ECORE + _EBIG, _ECO - _NCO)] = cor + _NBIG
        out[pl.ds(_ECORE + _EBIG + _ECO - _NCO, _NCO)] = arc_off


def _edges(vei, hei, coei, cotei):
    return pl.pallas_call(
        _edges_body,
        out_shape=[
            jax.ShapeDtypeStruct((2, _EBIG), jnp.int32),
            jax.ShapeDtypeStruct((2, _EBIG), jnp.int32),
            jax.ShapeDtypeStruct((2, _ECO), jnp.int32),
            jax.ShapeDtypeStruct((2, _ECO), jnp.int32),
            jax.ShapeDtypeStruct((2 * _ECORE,), jnp.int32),
            jax.ShapeDtypeStruct((2 * _ECORE,), jnp.int32),
        ],
    )(vei, hei, coei, cotei)


def _bigmm_body(vh_ref, hht_ref, pa_ref, pb_ref, p2_ref):
    t = jnp.dot(vh_ref[...], hht_ref[...], preferred_element_type=jnp.float32)
    pa_ref[...] = t
    pb_ref[...] = t
    p2_ref[...] = t + t


def _bigmm(vh, hht):
    n = vh.shape[0]
    m = hht.shape[1]
    tm = 256
    return pl.pallas_call(
        _bigmm_body,
        grid=(n // tm,),
        in_specs=[
            pl.BlockSpec((tm, _D), lambda i: (i, 0)),
            pl.BlockSpec((_D, m), lambda i: (0, 0)),
        ],
        out_specs=[
            pl.BlockSpec((tm, m), lambda i: (i, 0)),
            pl.BlockSpec((tm, m), lambda i: (i, 0)),
            pl.BlockSpec((tm, m), lambda i: (i, 0)),
        ],
        out_shape=[
            jax.ShapeDtypeStruct((n, m), jnp.float32),
            jax.ShapeDtypeStruct((n, m), jnp.float32),
            jax.ShapeDtypeStruct((n, m), jnp.float32),
        ],
    )(vh, hht)


@functools.cache
def _make_alpha_kernel():
    mesh = plsc.VectorSubcoreMesh(core_axis_name="c", subcore_axis_name="s")
    return functools.partial(
        pl.kernel,
        mesh=mesh,
        compiler_params=pltpu.CompilerParams(needs_layout_passes=False),
        out_type=jax.ShapeDtypeStruct((2 * _ECORE * _H,), jnp.float32),
        scratch_types=[
            pltpu.VMEM((_EPT,), jnp.int32),
            pltpu.VMEM((_EPT,), jnp.int32),
            pltpu.VMEM((_NT * 2 * _H,), jnp.float32),
            pltpu.VMEM((_DEN,), jnp.float32),
            pltpu.VMEM((_EPT,), jnp.float32),
            pltpu.VMEM((_RSEG,), jnp.float32),
            pltpu.VMEM_SHARED((_NSUB * _DEN,), jnp.float32),
            pltpu.VMEM_SHARED((_DEN,), jnp.float32),
        ],
    )(_alpha_body)


def _alpha_body(src_hbm, dst_hbm, tab_hbm, out_hbm,
                src_v, dst_v, tab_v, den_v, out3_v, tmp_v, stage, final_sh):
    c = lax.axis_index("c")
    s = lax.axis_index("s")
    ebase = c * _ECORE + s * _EPT
    pltpu.sync_copy(src_hbm.at[pl.ds(ebase, _EPT)], src_v)
    pltpu.sync_copy(dst_hbm.at[pl.ds(ebase, _EPT)], dst_v)
    for h in range(2 * _H):
        pltpu.sync_copy(tab_hbm.at[h, pl.ds(c * _NT, _NT)],
                        tab_v.at[pl.ds(h * _NT, _NT)])

    zero16 = jnp.zeros((16,), jnp.float32)

    @plsc.parallel_loop(0, _DEN // 16, unroll=4)
    def _zero(i):
        den_v[pl.ds(i * 16, 16)] = zero16

    @plsc.parallel_loop(0, _NCHUNK, unroll=2)
    def _pass1(i):
        off = pl.multiple_of(i * 16, 16)
        isrc = src_v[pl.ds(off, 16)]
        idst = dst_v[pl.ds(off, 16)]
        i3d = idst * _H
        for h in range(_H):
            a = (plsc.load_gather(tab_v, [isrc + h * _NT])
                 + plsc.load_gather(tab_v, [idst + (_H + h) * _NT]))
            a = jnp.where(a >= 0.0, a, 0.2 * a)
            plsc.addupdate_scatter(den_v, [i3d + h], jnp.exp(a))

    pltpu.sync_copy(den_v, stage.at[pl.ds(s * _DEN, _DEN)])
    plsc.subcore_barrier()

    rbase = s * _RSEG

    def _accrow(r, _):
        @pl.when(r != s)
        def _():
            pltpu.sync_copy(stage.at[pl.ds(r * _DEN + rbase, _RSEG)], tmp_v)

            @plsc.parallel_loop(0, _RSEG // 16, unroll=2)
            def _accvec(j):
                off = pl.multiple_of(j * 16, 16)
                den_v[pl.ds(rbase + off, 16)] = (
                    den_v[pl.ds(rbase + off, 16)] + tmp_v[pl.ds(off, 16)])
        return 0
    lax.fori_loop(0, _NSUB, _accrow, 0)

    @plsc.parallel_loop(0, _RSEG // 16, unroll=2)
    def _recip(j):
        off = pl.multiple_of(j * 16, 16)
        den_v[pl.ds(rbase + off, 16)] = 1.0 / (
            den_v[pl.ds(rbase + off, 16)] + 1e-16)

    pltpu.sync_copy(den_v.at[pl.ds(rbase, _RSEG)],
                    final_sh.at[pl.ds(rbase, _RSEG)])
    plsc.subcore_barrier()
    pltpu.sync_copy(final_sh, den_v)

    for h in range(_H):
        @plsc.parallel_loop(0, _NCHUNK, unroll=2)
        def _pass2(i):
            off = pl.multiple_of(i * 16, 16)
            isrc = src_v[pl.ds(off, 16)]
            idst = dst_v[pl.ds(off, 16)]
            a = (plsc.load_gather(tab_v, [isrc + h * _NT])
                 + plsc.load_gather(tab_v, [idst + (_H + h) * _NT]))
            a = jnp.where(a >= 0.0, a, 0.2 * a)
            rden = plsc.load_gather(den_v, [idst * _H + h])
            out3_v[pl.ds(off, 16)] = jnp.exp(a) * rden

        pltpu.sync_copy(
            out3_v,
            out_hbm.at[pl.ds((c * _H + h) * _ECORE + s * _EPT, _EPT)])


def _att_mat(att_src, att_dst):
    e8 = jnp.eye(8, dtype=jnp.float32)
    a_s = jnp.einsum("hd,hk->hdk", att_src[0], e8[:_H]).reshape(_H * _D, 8)
    a_d = jnp.einsum("hd,hk->hdk", att_dst[0],
                     e8[_H:2 * _H]).reshape(_H * _D, 8)
    return a_s + a_d


def kernel(virus_data, host_data, coexistence_data, virus_edge_index,
           host_edge_index, coexistence_edge_index, coexistence_edge_index_t,
           virus_edge_weight, host_edge_weight,
           W_gat_v, att_src_v, att_dst_v, b_gat_v,
           W_gat_h, att_src_h, att_dst_h, b_gat_h,
           W_gat_vh, att_src_vh, att_dst_vh, b_gat_vh,
           W_gat_hv, att_src_hv, att_dst_hv, b_gat_hv,
           W_lin_v, b_lin_v, W_lin_h, b_lin_h,
           bn_gamma, bn_beta, bn_mean, bn_var):
    scale = bn_gamma / jnp.sqrt(bn_var + 1e-5)

    am_v = _att_mat(att_src_v, att_dst_v)
    am_h = _att_mat(att_src_h, att_dst_h)
    am_vh = _att_mat(att_src_vh, att_dst_vh)
    am_hv = _att_mat(att_src_hv, att_dst_hv)
    vh, hht, tabt = _prep(
        virus_data, host_data, coexistence_data,
        W_gat_v, am_v, W_gat_h, am_h, W_gat_vh, am_vh, W_gat_hv, am_hv,
        W_lin_v, W_lin_h, scale[None, :], b_lin_v[None, :],
        b_lin_h[None, :], bn_mean[None, :], bn_beta[None, :])

    ei_v, ei_h, ei_vhv, ei_vhh, src_all, dst_all = _edges(
        virus_edge_index, host_edge_index, coexistence_edge_index,
        coexistence_edge_index_t)

    alpha_flat = _make_alpha_kernel()(src_all, dst_all, tabt)

    def _planes(core, lo, hi):
        return jnp.stack(
            [alpha_flat[(core * _H + h) * _ECORE + lo:
                        (core * _H + h) * _ECORE + hi] for h in range(_H)],
            axis=1)

    alpha_v = _planes(0, 0, _EBIG)
    alpha_vhv = _planes(0, _EBIG, _ECORE)
    alpha_h = _planes(1, 0, _EBIG)
    alpha_vhh = _planes(1, _EBIG, _ECORE)

    P, Pb, P2 = _bigmm(vh, hht)

    return (P, Pb, P2, (ei_v, alpha_v), (ei_h, alpha_h),
            (ei_vhv, alpha_vhv), (ei_vhh, alpha_vhh))


# speedup vs baseline: 6.0639x; 1.0015x over previous
import functools

import jax
import jax.numpy as jnp
from jax import lax
from jax.experimental import pallas as pl
from jax.experimental.pallas import tpu as pltpu
from jax.experimental.pallas import tpu_sc as plsc

_H = 3
_D = 128
_NBIG = 4096
_NCO = 512
_NT = _NBIG + _NCO
_EBIG = 262144 + _NBIG
_ECO = 16384 + _NCO
_ECORE = _EBIG + _ECO
_NSUB = 16
_EPT = _ECORE // _NSUB
_NCHUNK = _EPT // 16
_DEN = _NT * _H
_RSEG = _DEN // _NSUB


def _prep_body(vd, hd, co, wgv, av_m, wgh, ah_m, wgvh, avh_m,
               wghv, ahv_m, wlv, wlh, sc, blv, blh, bmean, bbeta,
               vh_out, hht_out, tabt):
    def logits(xv, wg, a_m, col):
        wab = jnp.dot(wg[...], a_m[...], preferred_element_type=jnp.float32)
        t = jnp.dot(xv, wab, preferred_element_type=jnp.float32)
        tabt[:, pl.ds(col, xv.shape[0])] = t.T

    logits(vd[...], wgv, av_m, 0)
    logits(co[...], wgvh, avh_m, _NBIG)
    logits(hd[...], wgh, ah_m, _NT)
    logits(co[...].T, wghv, ahv_m, _NT + _NBIG)

    scv = sc[...]
    sv = scv * (blv[...] - bmean[...]) + bbeta[...]
    sh = scv * (blh[...] - bmean[...]) + bbeta[...]
    t = (jnp.dot(vd[...], wlv[...] * scv,
                 preferred_element_type=jnp.float32) + sv)
    vh_out[...] = jnp.where(t >= 0.0, t, 0.01 * t)
    u = (jnp.dot(hd[...], wlh[...] * scv,
                 preferred_element_type=jnp.float32) + sh)
    hht_out[...] = jnp.where(u >= 0.0, u, 0.01 * u).T


def _prep(vd, hd, co, wgv, av_m, wgh, ah_m, wgvh, avh_m, wghv, ahv_m,
          wlv, wlh, sc, blv, blh, bmean, bbeta):
    n = vd.shape[0]
    return pl.pallas_call(
        _prep_body,
        out_shape=[
            jax.ShapeDtypeStruct((n, _D), jnp.float32),
            jax.ShapeDtypeStruct((_D, n), jnp.float32),
            jax.ShapeDtypeStruct((8, 2 * _NT), jnp.float32),
        ],
    )(vd, hd, co, wgv, av_m, wgh, ah_m, wgvh, avh_m, wghv, ahv_m,
      wlv, wlh, sc, blv, blh, bmean, bbeta)


def _edges_body(vei, hei, coei, cotei,
                eiv, eih, eivhv, eivhh, src_all, dst_all):
    arb_v = lax.broadcasted_iota(jnp.int32, (1, _NBIG), 1)[0]
    arc_v = lax.broadcasted_iota(jnp.int32, (1, _NCO), 1)[0]
    arc_off = arc_v + _NBIG
    for row in (0, 1):
        vr = vei[row, :]
        hr = hei[row, :]
        cor = coei[row, :]
        cotr = cotei[row, :]
        eiv[row, pl.ds(0, _EBIG - _NBIG)] = vr
        eiv[row, pl.ds(_EBIG - _NBIG, _NBIG)] = arb_v
        eih[row, pl.ds(0, _EBIG - _NBIG)] = hr
        eih[row, pl.ds(_EBIG - _NBIG, _NBIG)] = arb_v
        eivhv[row, pl.ds(0, _ECO - _NCO)] = cotr
        eivhv[row, pl.ds(_ECO - _NCO, _NCO)] = arc_v
        eivhh[row, pl.ds(0, _ECO - _NCO)] = cor
        eivhh[row, pl.ds(_ECO - _NCO, _NCO)] = arc_v
        out = src_all if row == 0 else dst_all
        out[pl.ds(0, _EBIG - _NBIG)] = vr
        out[pl.ds(_EBIG - _NBIG, _NBIG)] = arb_v
        out[pl.ds(_EBIG, _ECO - _NCO)] = cotr + _NBIG
        out[pl.ds(_EBIG + _ECO - _NCO, _NCO)] = arc_off
        out[pl.ds(_ECORE, _EBIG - _NBIG)] = hr
        out[pl.ds(_ECORE + _EBIG - _NBIG, _NBIG)] = arb_v
        out[pl.ds(_ECORE + _EBIG, _ECO - _NCO)] = cor + _NBIG
        out[pl.ds(_ECORE + _EBIG + _ECO - _NCO, _NCO)] = arc_off


def _edges(vei, hei, coei, cotei):
    return pl.pallas_call(
        _edges_body,
        out_shape=[
            jax.ShapeDtypeStruct((2, _EBIG), jnp.int32),
            jax.ShapeDtypeStruct((2, _EBIG), jnp.int32),
            jax.ShapeDtypeStruct((2, _ECO), jnp.int32),
            jax.ShapeDtypeStruct((2, _ECO), jnp.int32),
            jax.ShapeDtypeStruct((2 * _ECORE,), jnp.int32),
            jax.ShapeDtypeStruct((2 * _ECORE,), jnp.int32),
        ],
    )(vei, hei, coei, cotei)


def _bigmm_body(vh_ref, hht_ref, pa_ref, pb_ref, p2_ref):
    t = jnp.dot(vh_ref[...], hht_ref[...], preferred_element_type=jnp.float32)
    pa_ref[...] = t
    pb_ref[...] = t
    p2_ref[...] = t + t


def _bigmm(vh, hht):
    n = vh.shape[0]
    m = hht.shape[1]
    tm = 256
    return pl.pallas_call(
        _bigmm_body,
        grid=(n // tm,),
        in_specs=[
            pl.BlockSpec((tm, _D), lambda i: (i, 0)),
            pl.BlockSpec((_D, m), lambda i: (0, 0)),
        ],
        out_specs=[
            pl.BlockSpec((tm, m), lambda i: (i, 0)),
            pl.BlockSpec((tm, m), lambda i: (i, 0)),
            pl.BlockSpec((tm, m), lambda i: (i, 0)),
        ],
        out_shape=[
            jax.ShapeDtypeStruct((n, m), jnp.float32),
            jax.ShapeDtypeStruct((n, m), jnp.float32),
            jax.ShapeDtypeStruct((n, m), jnp.float32),
        ],
    )(vh, hht)


@functools.cache
def _make_alpha_kernel():
    mesh = plsc.VectorSubcoreMesh(core_axis_name="c", subcore_axis_name="s")
    return functools.partial(
        pl.kernel,
        mesh=mesh,
        compiler_params=pltpu.CompilerParams(needs_layout_passes=False),
        out_type=jax.ShapeDtypeStruct((2 * _ECORE * _H,), jnp.float32),
        scratch_types=[
            pltpu.VMEM((_EPT,), jnp.int32),
            pltpu.VMEM((_EPT,), jnp.int32),
            pltpu.VMEM((_NT * 2 * _H,), jnp.float32),
            pltpu.VMEM((_DEN,), jnp.float32),
            pltpu.VMEM((_EPT,), jnp.float32),
            pltpu.VMEM((_RSEG,), jnp.float32),
            pltpu.VMEM_SHARED((_NSUB * _DEN,), jnp.float32),
            pltpu.VMEM_SHARED((_DEN,), jnp.float32),
        ],
    )(_alpha_body)


def _alpha_body(src_hbm, dst_hbm, tab_hbm, out_hbm,
                src_v, dst_v, tab_v, den_v, out3_v, tmp_v, stage, final_sh):
    c = lax.axis_index("c")
    s = lax.axis_index("s")
    ebase = c * _ECORE + s * _EPT
    pltpu.sync_copy(src_hbm.at[pl.ds(ebase, _EPT)], src_v)
    pltpu.sync_copy(dst_hbm.at[pl.ds(ebase, _EPT)], dst_v)
    for h in range(2 * _H):
        pltpu.sync_copy(tab_hbm.at[h, pl.ds(c * _NT, _NT)],
                        tab_v.at[pl.ds(h * _NT, _NT)])

    zero16 = jnp.zeros((16,), jnp.float32)

    @plsc.parallel_loop(0, _DEN // 16, unroll=4)
    def _zero(i):
        den_v[pl.ds(i * 16, 16)] = zero16

    @plsc.parallel_loop(0, _NCHUNK, unroll=2)
    def _pass1(i):
        off = pl.multiple_of(i * 16, 16)
        isrc = src_v[pl.ds(off, 16)]
        idst = dst_v[pl.ds(off, 16)]
        i3d = idst * _H
        for h in range(_H):
            a = (plsc.load_gather(tab_v, [isrc + h * _NT])
                 + plsc.load_gather(tab_v, [idst + (_H + h) * _NT]))
            a = jnp.where(a >= 0.0, a, 0.2 * a)
            plsc.addupdate_scatter(den_v, [i3d + h], jnp.exp(a))

    pltpu.sync_copy(den_v, stage.at[pl.ds(s * _DEN, _DEN)])
    plsc.subcore_barrier()

    rbase = s * _RSEG

    def _accrow(r, _):
        @pl.when(r != s)
        def _():
            pltpu.sync_copy(stage.at[pl.ds(r * _DEN + rbase, _RSEG)], tmp_v)

            @plsc.parallel_loop(0, _RSEG // 16, unroll=2)
            def _accvec(j):
                off = pl.multiple_of(j * 16, 16)
                den_v[pl.ds(rbase + off, 16)] = (
                    den_v[pl.ds(rbase + off, 16)] + tmp_v[pl.ds(off, 16)])
        return 0
    lax.fori_loop(0, _NSUB, _accrow, 0)

    @plsc.parallel_loop(0, _RSEG // 16, unroll=2)
    def _recip(j):
        off = pl.multiple_of(j * 16, 16)
        den_v[pl.ds(rbase + off, 16)] = 1.0 / (
            den_v[pl.ds(rbase + off, 16)] + 1e-16)

    pltpu.sync_copy(den_v.at[pl.ds(rbase, _RSEG)],
                    final_sh.at[pl.ds(rbase, _RSEG)])
    plsc.subcore_barrier()
    pltpu.sync_copy(final_sh, den_v)

    for h in range(_H):
        @plsc.parallel_loop(0, _NCHUNK, unroll=2)
        def _pass2(i):
            off = pl.multiple_of(i * 16, 16)
            isrc = src_v[pl.ds(off, 16)]
            idst = dst_v[pl.ds(off, 16)]
            a = (plsc.load_gather(tab_v, [isrc + h * _NT])
                 + plsc.load_gather(tab_v, [idst + (_H + h) * _NT]))
            a = jnp.where(a >= 0.0, a, 0.2 * a)
            rden = plsc.load_gather(den_v, [idst * _H + h])
            out3_v[pl.ds(off, 16)] = jnp.exp(a) * rden

        pltpu.sync_copy(
            out3_v,
            out_hbm.at[pl.ds((c * _H + h) * _ECORE + s * _EPT, _EPT)])


def _att_mat(att_src, att_dst):
    e8 = jnp.eye(8, dtype=jnp.float32)
    a_s = jnp.einsum("hd,hk->hdk", att_src[0], e8[:_H]).reshape(_H * _D, 8)
    a_d = jnp.einsum("hd,hk->hdk", att_dst[0],
                     e8[_H:2 * _H]).reshape(_H * _D, 8)
    return a_s + a_d


def kernel(virus_data, host_data, coexistence_data, virus_edge_index,
           host_edge_index, coexistence_edge_index, coexistence_edge_index_t,
           virus_edge_weight, host_edge_weight,
           W_gat_v, att_src_v, att_dst_v, b_gat_v,
           W_gat_h, att_src_h, att_dst_h, b_gat_h,
           W_gat_vh, att_src_vh, att_dst_vh, b_gat_vh,
           W_gat_hv, att_src_hv, att_dst_hv, b_gat_hv,
           W_lin_v, b_lin_v, W_lin_h, b_lin_h,
           bn_gamma, bn_beta, bn_mean, bn_var):
    scale = bn_gamma / jnp.sqrt(bn_var + 1e-5)

    am_v = _att_mat(att_src_v, att_dst_v)
    am_h = _att_mat(att_src_h, att_dst_h)
    am_vh = _att_mat(att_src_vh, att_dst_vh)
    am_hv = _att_mat(att_src_hv, att_dst_hv)
    vh, hht, tabt = _prep(
        virus_data, host_data, coexistence_data,
        W_gat_v, am_v, W_gat_h, am_h, W_gat_vh, am_vh, W_gat_hv, am_hv,
        W_lin_v, W_lin_h, scale[None, :], b_lin_v[None, :],
        b_lin_h[None, :], bn_mean[None, :], bn_beta[None, :])

    ei_v, ei_h, ei_vhv, ei_vhh, src_all, dst_all = _edges(
        virus_edge_index, host_edge_index, coexistence_edge_index,
        coexistence_edge_index_t)

    alpha_flat = _make_alpha_kernel()(src_all, dst_all, tabt)

    def _planes(core, lo, hi):
        return jnp.stack(
            [alpha_flat[(core * _H + h) * _ECORE + lo:
                        (core * _H + h) * _ECORE + hi] for h in range(_H)],
            axis=1)

    alpha_v = _planes(0, 0, _EBIG)
    alpha_vhv = _planes(0, _EBIG, _ECORE)
    alpha_h = _planes(1, 0, _EBIG)
    alpha_vhh = _planes(1, _EBIG, _ECORE)

    P, Pb, P2 = _bigmm(vh, hht)

    return (P, Pb, P2, (ei_v, alpha_v), (ei_h, alpha_h),
            (ei_vhv, alpha_vhv), (ei_vhh, alpha_vhh))
